# baseline (speedup 1.0000x reference)
import functools

import jax
import jax.numpy as jnp
from jax import lax
from jax.experimental import pallas as pl
from jax.experimental.pallas import tpu as pltpu
from jax.experimental.pallas import tpu_sc as plsc

_NC = 2
_NS = 16
_NW = _NC * _NS

_CH = 128
_NBUF = 4
_L = 16


def _make_gather(v2, d2, b_total):
    assert b_total % _NW == 0
    b_per_w = b_total // _NW
    assert b_per_w % _CH == 0
    n_ch = b_per_w // _CH
    n_pad = n_ch + (-n_ch) % 8
    d = d2 // 2
    n_it = n_ch + (-n_ch) % _NBUF
    n_grp = n_it // _NBUF
    chunk_elems = _CH * d

    mesh = plsc.VectorSubcoreMesh(core_axis_name="c", subcore_axis_name="s")

    @functools.partial(
        pl.kernel,
        mesh=mesh,
        compiler_params=pltpu.CompilerParams(needs_layout_passes=False),
        out_type=jax.ShapeDtypeStruct((b_total * d,), jnp.float32),
        scratch_types=[
            pltpu.VMEM((n_pad, _CH), jnp.int32),
            pltpu.VMEM((n_pad, _CH), jnp.int32),
            [pltpu.VMEM((_CH, d2), jnp.float32) for _ in range(_NBUF)],
            [pltpu.VMEM((chunk_elems,), jnp.float32) for _ in range(_NBUF)],
            pltpu.SemaphoreType.DMA((_NBUF,)),
            pltpu.SemaphoreType.DMA((_NBUF,)),
        ],
    )
    def gather_kernel(
        idx_hbm, table_hbm, out_hbm, idx_v, ridx_v, pair_v, outb_v, gsem, ssem
    ):
        wid = lax.axis_index("s") * _NC + lax.axis_index("c")
        pltpu.sync_copy(idx_hbm.at[pl.ds(wid * n_pad, n_pad)], idx_v)

        def compute_ridx(j):
            for g in range(_CH // _L):
                sl = pl.ds(g * _L, _L)
                ridx_v[j, sl] = idx_v[j, sl] >> 1

        def gather(j, b):
            return pltpu.make_async_copy(
                table_hbm.at[ridx_v.at[j]], pair_v[b], gsem.at[b]
            )

        def store(j, b):
            return pltpu.make_async_copy(
                outb_v[b],
                out_hbm.at[pl.ds((wid * n_ch + j) * chunk_elems, chunk_elems)],
                ssem.at[b],
            )

        def select(j, b):
            lanes = lax.iota(jnp.int32, _L)
            for g in range(_CH // _L):
                row = lanes + (g * _L)
                col0 = (idx_v[j, pl.ds(g * _L, _L)] & 1) << 6
                obase0 = row << 6
                for c in range(d):
                    val = plsc.load_gather(pair_v[b], [row, col0 + c])
                    plsc.store_scatter(outb_v[b], [obase0 + c], val)

        for b in range(_NBUF):
            compute_ridx(b)
            gather(b, b).start()

        def group(g, _):
            for b in range(_NBUF):
                j = g * _NBUF + b

                @pl.when(j < n_ch)
                def _():
                    gather(j, b).wait()

                    @pl.when(j >= _NBUF)
                    def _():
                        store(j - _NBUF, b).wait()

                    select(j, b)
                    store(j, b).start()
                    jn = j + _NBUF

                    @pl.when(jn < n_ch)
                    def _():
                        compute_ridx(jn)
                        gather(jn, b).start()

            return ()

        lax.fori_loop(0, n_grp, group, ())

        for j in range(n_ch - _NBUF, n_ch):
            store(j, j % _NBUF).wait()

    return gather_kernel


def kernel(stroke_orders, embedding_table):
    batch, hist = stroke_orders.shape
    v, d = embedding_table.shape
    b_total = batch * hist
    table2 = embedding_table.reshape(v // 2, 2 * d)
    n_ch = b_total // (_NW * _CH)
    n_pad = n_ch + (-n_ch) % 8
    idx = stroke_orders.astype(jnp.int32).reshape(_NW, n_ch, _CH)
    idx = jnp.pad(idx, ((0, 0), (0, n_pad - n_ch), (0, 0)))
    idx = idx.reshape(_NW * n_pad, _CH)
    out = _make_gather(v // 2, 2 * d, b_total)(idx, table2)
    return out.reshape(batch, hist, d)


# speedup vs baseline: 1.2187x; 1.2187x over previous
import functools

import jax
import jax.numpy as jnp
from jax import lax
from jax.experimental import pallas as pl
from jax.experimental.pallas import tpu as pltpu
from jax.experimental.pallas import tpu_sc as plsc

_NC = 2
_NS = 16
_NW = _NC * _NS

_CH = 128
_NBUF = 4
_L = 16


def _make_gather(v, d, batch, hist):
    b_total = batch * hist
    assert b_total % _NW == 0
    b_per_w = b_total // _NW
    assert b_per_w % _CH == 0
    n_ch = b_per_w // _CH
    blocks_per_h = batch // _CH
    n_it = n_ch + (-n_ch) % _NBUF
    n_grp = n_it // _NBUF

    mesh = plsc.VectorSubcoreMesh(core_axis_name="c", subcore_axis_name="s")

    @functools.partial(
        pl.kernel,
        mesh=mesh,
        compiler_params=pltpu.CompilerParams(
            use_tc_tiling_on_sc=False, needs_layout_passes=False
        ),
        out_type=jax.ShapeDtypeStruct((hist * d, batch), jnp.float32),
        scratch_types=[
            pltpu.VMEM((n_ch, _CH), jnp.int32),
            [pltpu.VMEM((_CH, d), jnp.float32) for _ in range(_NBUF)],
            [pltpu.VMEM((d, _CH), jnp.float32) for _ in range(_NBUF)],
            pltpu.SemaphoreType.DMA((_NBUF,)),
            pltpu.SemaphoreType.DMA((_NBUF,)),
        ],
    )
    def gather_kernel(
        idx_hbm, table_hbm, out_hbm, idx_v, rows_v, outt_v, gsem, ssem
    ):
        wid = lax.axis_index("s") * _NC + lax.axis_index("c")
        pltpu.sync_copy(idx_hbm.at[wid], idx_v)

        def gather(j, b):
            return pltpu.make_async_copy(
                table_hbm.at[idx_v.at[j]], rows_v[b], gsem.at[b]
            )

        def store(j, b):
            c = wid * n_ch + j
            h = c // blocks_per_h
            b0 = (c % blocks_per_h) * _CH
            return pltpu.make_async_copy(
                outt_v[b],
                out_hbm.at[pl.ds(h * d, d), pl.ds(b0, _CH)],
                ssem.at[b],
            )

        def transpose(b):
            lanes = lax.iota(jnp.int32, _L)
            for g in range(_CH // _L):
                kvec = lanes + (g * _L)
                for j in range(d):
                    jvec = jnp.full((_L,), j, jnp.int32)
                    val = plsc.load_gather(rows_v[b], [kvec, jvec])
                    outt_v[b][j, pl.ds(g * _L, _L)] = val

        for b in range(_NBUF):
            gather(b, b).start()

        def group(g, _):
            for b in range(_NBUF):
                j = g * _NBUF + b

                @pl.when(j < n_ch)
                def _():
                    gather(j, b).wait()

                    @pl.when(j >= _NBUF)
                    def _():
                        store(j - _NBUF, b).wait()

                    transpose(b)
                    store(j, b).start()
                    jn = j + _NBUF

                    @pl.when(jn < n_ch)
                    def _():
                        gather(jn, b).start()

            return ()

        lax.fori_loop(0, n_grp, group, ())

        for j in range(n_ch - _NBUF, n_ch):
            store(j, j % _NBUF).wait()

    return gather_kernel


def kernel(stroke_orders, embedding_table):
    batch, hist = stroke_orders.shape
    v, d = embedding_table.shape
    b_total = batch * hist
    idxt = stroke_orders.astype(jnp.int32).T.reshape(
        _NW, b_total // (_NW * _CH), _CH
    )
    out = _make_gather(v, d, batch, hist)(idxt, embedding_table)
    return jnp.transpose(out.reshape(hist, d, batch), (2, 0, 1))


# speedup vs baseline: 1.5842x; 1.2999x over previous
import functools

import jax
import jax.numpy as jnp
from jax import lax
from jax.experimental import pallas as pl
from jax.experimental.pallas import tpu as pltpu
from jax.experimental.pallas import tpu_sc as plsc

_NC = 2
_NS = 16
_NW = _NC * _NS

_CH = 128
_NBUF = 5


def _make_gather(v, d, b_total):
    assert b_total % _NW == 0
    b_per_w = b_total // _NW
    assert b_per_w % _CH == 0
    n_ch = b_per_w // _CH
    assert n_ch % _NBUF == 0 and n_ch // _NBUF >= 2
    n_grp = n_ch // _NBUF

    mesh = plsc.VectorSubcoreMesh(core_axis_name="c", subcore_axis_name="s")

    @functools.partial(
        pl.kernel,
        mesh=mesh,
        compiler_params=pltpu.CompilerParams(use_tc_tiling_on_sc=False),
        out_type=jax.ShapeDtypeStruct((_NW, n_ch, _CH, d), jnp.float32),
        scratch_types=[
            pltpu.VMEM((n_ch, _CH), jnp.int32),
            pltpu.VMEM((_NBUF, _CH, d), jnp.float32),
            pltpu.SemaphoreType.DMA((_NBUF,)),
            pltpu.SemaphoreType.DMA((_NBUF,)),
        ],
    )
    def gather_kernel(idx_hbm, table_hbm, out_hbm, idx_v, rows_v, gsem, ssem):
        wid = lax.axis_index("s") * _NC + lax.axis_index("c")
        pltpu.sync_copy(idx_hbm.at[wid], idx_v)

        for b in range(_NBUF):
            pltpu.async_copy(table_hbm.at[idx_v.at[b]], rows_v.at[b], gsem.at[b])

        def group(g, _):
            for b in range(_NBUF):
                j = g * _NBUF + b
                pltpu.make_async_copy(
                    table_hbm.at[idx_v.at[b]], rows_v.at[b], gsem.at[b]
                ).wait()
                pltpu.async_copy(rows_v.at[b], out_hbm.at[wid, j], ssem.at[b])
                jn = j + _NBUF

                @pl.when(jn < n_ch)
                def _():
                    pltpu.make_async_copy(
                        rows_v.at[b], out_hbm.at[wid, j], ssem.at[b]
                    ).wait()
                    pltpu.async_copy(
                        table_hbm.at[idx_v.at[jn]], rows_v.at[b], gsem.at[b]
                    )

            return ()

        lax.fori_loop(0, n_grp, group, ())

        for b in range(_NBUF):
            pltpu.make_async_copy(
                rows_v.at[b], out_hbm.at[wid, n_ch - _NBUF + b], ssem.at[b]
            ).wait()

    return gather_kernel


def kernel(stroke_orders, embedding_table):
    batch, hist = stroke_orders.shape
    v, d = embedding_table.shape
    b_total = batch * hist
    table_flat = lax.optimization_barrier(embedding_table.reshape(v * d))
    table_lin = table_flat.reshape(v, d)
    idx = stroke_orders.astype(jnp.int32).reshape(_NW, b_total // (_NW * _CH), _CH)
    out = _make_gather(v, d, b_total)(idx, table_lin)
    return out.reshape(batch, hist, d)


# speedup vs baseline: 1.6754x; 1.0576x over previous
import functools

import jax
import jax.numpy as jnp
from jax import lax
from jax.experimental import pallas as pl
from jax.experimental.pallas import tpu as pltpu
from jax.experimental.pallas import tpu_sc as plsc

_NC = 2
_NS = 16
_NW = _NC * _NS

_CH = 128
_NBUF = 5


def _make_gather(v, d, b_total):
    assert b_total % _NW == 0
    b_per_w = b_total // _NW
    assert b_per_w % _CH == 0
    n_ch = b_per_w // _CH
    assert n_ch % _NBUF == 0 and n_ch // _NBUF >= 2
    n_grp = n_ch // _NBUF

    mesh = plsc.VectorSubcoreMesh(core_axis_name="c", subcore_axis_name="s")

    @functools.partial(
        pl.kernel,
        mesh=mesh,
        compiler_params=pltpu.CompilerParams(use_tc_tiling_on_sc=False),
        out_type=jax.ShapeDtypeStruct((_NW, n_ch, _CH, d), jnp.float32),
        scratch_types=[
            pltpu.VMEM((n_ch, _CH), jnp.int32),
            pltpu.VMEM((_NBUF, _CH, 2 * d), jnp.float32),
            pltpu.SemaphoreType.DMA((_NBUF,)),
            pltpu.SemaphoreType.DMA((_NBUF,)),
        ],
    )
    def gather_kernel(idx_hbm, table_hbm, out_hbm, idx_v, rows_v, gsem, ssem):
        wid = lax.axis_index("s") * _NC + lax.axis_index("c")
        pltpu.sync_copy(idx_hbm.at[wid], idx_v)

        for b in range(_NBUF):
            pltpu.async_copy(table_hbm.at[idx_v.at[b]], rows_v.at[b], gsem.at[b])

        def group(g, _):
            for b in range(_NBUF):
                j = g * _NBUF + b
                pltpu.make_async_copy(
                    table_hbm.at[idx_v.at[b]], rows_v.at[b], gsem.at[b]
                ).wait()
                pltpu.async_copy(rows_v.at[b, :, pl.ds(0, d)], out_hbm.at[wid, j], ssem.at[b])
                jn = j + _NBUF

                @pl.when(jn < n_ch)
                def _():
                    pltpu.make_async_copy(
                        rows_v.at[b, :, pl.ds(0, d)], out_hbm.at[wid, j],
                        ssem.at[b],
                    ).wait()
                    pltpu.async_copy(
                        table_hbm.at[idx_v.at[jn]], rows_v.at[b], gsem.at[b]
                    )

            return ()

        lax.fori_loop(0, n_grp, group, ())

        for b in range(_NBUF):
            pltpu.make_async_copy(
                rows_v.at[b, :, pl.ds(0, d)],
                out_hbm.at[wid, n_ch - _NBUF + b], ssem.at[b]
            ).wait()

    return gather_kernel


def kernel(stroke_orders, embedding_table):
    batch, hist = stroke_orders.shape
    v, d = embedding_table.shape
    b_total = batch * hist
    table_lin = jnp.pad(embedding_table, ((0, 0), (0, d)))
    idx = stroke_orders.astype(jnp.int32).reshape(_NW, b_total // (_NW * _CH), _CH)
    out = _make_gather(v, d, b_total)(idx, table_lin)
    return out.reshape(batch, hist, d)


# speedup vs baseline: 1.6779x; 1.0015x over previous
import functools

import jax
import jax.numpy as jnp
from jax import lax
from jax.experimental import pallas as pl
from jax.experimental.pallas import tpu as pltpu
from jax.experimental.pallas import tpu_sc as plsc

_NC = 2
_NS = 16
_NW = _NC * _NS

_CH = 128
_NBUF = 5


def _make_gather(v, d, b_total):
    assert b_total % _NW == 0
    b_per_w = b_total // _NW
    assert b_per_w % _CH == 0
    n_ch = b_per_w // _CH
    assert n_ch % _NBUF == 0 and n_ch // _NBUF >= 2
    n_grp = n_ch // _NBUF

    mesh = plsc.VectorSubcoreMesh(core_axis_name="c", subcore_axis_name="s")

    @functools.partial(
        pl.kernel,
        mesh=mesh,
        compiler_params=pltpu.CompilerParams(use_tc_tiling_on_sc=False),
        out_type=jax.ShapeDtypeStruct((b_total, d), jnp.float32),
        scratch_types=[
            pltpu.VMEM((n_ch, _CH), jnp.int32),
            pltpu.VMEM((_NBUF, _CH, 2 * d), jnp.float32),
            pltpu.SemaphoreType.DMA((_NBUF,)),
            pltpu.SemaphoreType.DMA((_NBUF,)),
        ],
    )
    def gather_kernel(idx_hbm, table_hbm, out_hbm, idx_v, rows_v, gsem, ssem):
        wid = lax.axis_index("s") * _NC + lax.axis_index("c")
        pltpu.sync_copy(idx_hbm.at[wid], idx_v)

        for b in range(_NBUF):
            pltpu.async_copy(table_hbm.at[idx_v.at[b]], rows_v.at[b], gsem.at[b])

        def group(g, _):
            for b in range(_NBUF):
                j = g * _NBUF + b
                pltpu.make_async_copy(
                    table_hbm.at[idx_v.at[b]], rows_v.at[b], gsem.at[b]
                ).wait()
                pltpu.async_copy(rows_v.at[b, :, pl.ds(0, d)], out_hbm.at[pl.ds((wid * n_ch + j) * _CH, _CH)], ssem.at[b])
                jn = j + _NBUF

                @pl.when(jn < n_ch)
                def _():
                    pltpu.make_async_copy(
                        rows_v.at[b, :, pl.ds(0, d)],
                        out_hbm.at[pl.ds((wid * n_ch + j) * _CH, _CH)],
                        ssem.at[b],
                    ).wait()
                    pltpu.async_copy(
                        table_hbm.at[idx_v.at[jn]], rows_v.at[b], gsem.at[b]
                    )

            return ()

        lax.fori_loop(0, n_grp, group, ())

        for b in range(_NBUF):
            pltpu.make_async_copy(
                rows_v.at[b, :, pl.ds(0, d)],
                out_hbm.at[pl.ds((wid * n_ch + n_ch - _NBUF + b) * _CH, _CH)],
                ssem.at[b],
            ).wait()

    return gather_kernel


def kernel(stroke_orders, embedding_table):
    batch, hist = stroke_orders.shape
    v, d = embedding_table.shape
    b_total = batch * hist
    table_lin = jnp.pad(embedding_table, ((0, 0), (0, d)))
    idx = stroke_orders.astype(jnp.int32).reshape(_NW, b_total // (_NW * _CH), _CH)
    out = _make_gather(v, d, b_total)(idx, table_lin)
    return out.reshape(batch, hist, d)
